# baseline (speedup 1.0000x reference)
import functools

import jax
import jax.numpy as jnp
from jax import lax
from jax.experimental import pallas as pl
from jax.experimental.pallas import tpu as pltpu
from jax.experimental.pallas import tpu_sc as plsc

_NEG = 0.2
_G = 64


def _leaky(t):
    return jnp.where(t >= 0, t, _NEG * t)


def _sc_degree(dst2, npad):
    nw, ept = dst2.shape
    mesh = plsc.VectorSubcoreMesh(core_axis_name="c", subcore_axis_name="s")

    @functools.partial(
        pl.kernel,
        mesh=mesh,
        compiler_params=pltpu.CompilerParams(needs_layout_passes=False),
        out_type=jax.ShapeDtypeStruct((nw, npad), jnp.float32),
        scratch_types=[
            pltpu.VMEM((ept,), jnp.int32),
            pltpu.VMEM((npad,), jnp.float32),
        ],
    )
    def deg_kernel(dst_hbm, out_hbm, dst_v, deg_v):
        c = lax.axis_index("c")
        s = lax.axis_index("s")
        wid = s * 2 + c
        pltpu.sync_copy(dst_hbm.at[wid], dst_v)

        zero16 = jnp.zeros((16,), jnp.float32)

        def zbody(ri, carry):
            deg_v[pl.ds(ri * 16, 16)] = zero16
            return carry

        lax.fori_loop(0, npad // 16, zbody, 0)

        one16 = jnp.ones((16,), jnp.float32)

        def body(gi, carry):
            idx = dst_v[pl.ds(gi * 16, 16)]
            plsc.addupdate_scatter(deg_v, [idx], one16)
            return carry

        lax.fori_loop(0, ept // 16, body, 0)
        pltpu.sync_copy(deg_v, out_hbm.at[wid])

    return deg_kernel(dst2)


def _sc_edge_agg(y, src3, dst3, zeros_nd, slow_core, ngroups_slow):
    n, d = y.shape
    nw, c_chunks, k = src3.shape
    npad = zeros_nd.shape[0]
    rpt = npad // 16
    gsz = 32
    npairs = gsz // 2
    ngroups_fast = c_chunks // gsz
    mesh = plsc.VectorSubcoreMesh(core_axis_name="c", subcore_axis_name="s")

    @functools.partial(
        pl.kernel,
        mesh=mesh,
        out_type=jax.ShapeDtypeStruct((2 * npad, d), jnp.float32),
        scratch_types=[
            pltpu.VMEM((gsz, k), jnp.int32),
            pltpu.VMEM((gsz, k), jnp.int32),
            pltpu.VMEM((2, k, d), jnp.float32),
            pltpu.VMEM_SHARED((npad, d), jnp.float32),
            pltpu.SemaphoreType.DMA,
            pltpu.SemaphoreType.DMA,
            pltpu.SemaphoreType.DMA,
            pltpu.SemaphoreType.DMA,
        ],
    )
    def agg_kernel(y_hbm, src_hbm, dst_hbm, z_hbm, out_hbm,
                   src_v, dst_v, rows_v, agg_sh, g0, g1, s0, s1):
        c = lax.axis_index("c")
        s = lax.axis_index("s")
        wid = s * 2 + c
        ngroups_me = lax.select(c == slow_core,
                                jnp.int32(ngroups_slow),
                                jnp.int32(ngroups_fast))
        pltpu.sync_copy(z_hbm.at[pl.ds(s * rpt, rpt)],
                        agg_sh.at[pl.ds(s * rpt, rpt)])
        plsc.subcore_barrier()

        def gather(ci, buf, sem):
            pltpu.async_copy(y_hbm.at[src_v.at[ci]], rows_v.at[buf], sem)

        def wait_gather(buf, sem):
            pltpu.make_async_copy(y_hbm.at[pl.ds(0, k)], rows_v.at[buf],
                                  sem).wait()

        def scatter(ci, buf, sem):
            pltpu.async_copy(rows_v.at[buf], agg_sh.at[dst_v.at[ci]], sem,
                             add=True)

        def wait_scatter(buf, sem):
            pltpu.make_async_copy(rows_v.at[buf], agg_sh.at[dst_v.at[0]],
                                  sem).wait()

        def group(g, carry):
            @pl.when(g > 0)
            def _():
                wait_scatter(0, s0)
                wait_scatter(1, s1)

            pltpu.sync_copy(src_hbm.at[wid, pl.ds(g * gsz, gsz)], src_v)
            pltpu.sync_copy(dst_hbm.at[wid, pl.ds(g * gsz, gsz)], dst_v)
            gather(0, 0, g0)
            gather(1, 1, g1)

            def body(p, carry2):
                c0 = 2 * p
                wait_gather(0, g0)
                scatter(c0, 0, s0)
                wait_gather(1, g1)
                scatter(c0 + 1, 1, s1)

                @pl.when(p < npairs - 1)
                def _():
                    wait_scatter(0, s0)
                    gather(c0 + 2, 0, g0)
                    wait_scatter(1, s1)
                    gather(c0 + 3, 1, g1)

                return carry2

            lax.fori_loop(0, npairs, body, 0)
            return carry

        lax.fori_loop(0, ngroups_me, group, 0)
        wait_scatter(0, s0)
        wait_scatter(1, s1)
        plsc.subcore_barrier()
        pltpu.sync_copy(agg_sh.at[pl.ds(s * rpt, rpt)],
                        out_hbm.at[pl.ds(c * npad + s * rpt, rpt)])

    return agg_kernel(y, src3, dst3, zeros_nd).reshape(2, npad, d)


def _dinv_from(dp):
    return lax.rsqrt(1.0 + jnp.sum(dp, axis=1))


def _tc_first(x, w1, degp, bn):
    n, d = x.shape
    h = w1.shape[1]

    def body(x_ref, w_ref, dp_ref, o_ref):
        dinv = _dinv_from(dp_ref[...])
        xw = jnp.dot(x_ref[...], w_ref[...], preferred_element_type=jnp.float32)
        o_ref[...] = xw * dinv[:, None]

    return pl.pallas_call(
        body,
        grid=(n // bn,),
        in_specs=[
            pl.BlockSpec((bn, d), lambda i: (i, 0)),
            pl.BlockSpec((d, h), lambda i: (0, 0)),
            pl.BlockSpec((bn, 32), lambda i: (i, 0)),
        ],
        out_specs=pl.BlockSpec((bn, h), lambda i: (i, 0)),
        out_shape=jax.ShapeDtypeStruct((n, h), jnp.float32),
    )(x, w1, degp)


def _tc_mid(y1, agg1, degp, b1, w2, bn):
    n, h = y1.shape

    def body(y_ref, a_ref, dp_ref, b_ref, w_ref, o_ref):
        dinv = _dinv_from(dp_ref[...])
        a = a_ref[...]
        t = (a[0] + a[1] + y_ref[...]) * dinv[:, None] + b_ref[...]
        hh = _leaky(t)
        o_ref[...] = jnp.dot(hh, w_ref[...],
                             preferred_element_type=jnp.float32) * dinv[:, None]

    return pl.pallas_call(
        body,
        grid=(n // bn,),
        in_specs=[
            pl.BlockSpec((bn, h), lambda i: (i, 0)),
            pl.BlockSpec((2, bn, h), lambda i: (0, i, 0)),
            pl.BlockSpec((bn, 32), lambda i: (i, 0)),
            pl.BlockSpec((1, h), lambda i: (0, 0)),
            pl.BlockSpec((h, h), lambda i: (0, 0)),
        ],
        out_specs=pl.BlockSpec((bn, h), lambda i: (i, 0)),
        out_shape=jax.ShapeDtypeStruct((n, h), jnp.float32),
    )(y1, agg1, degp, b1, w2)


def _tc_last(y2, agg2, degp, b2, wfc, bfc, batch3, bn):
    n, h = y2.shape
    grid = n // bn

    def body(y_ref, a_ref, dp_ref, b_ref, w_ref, bias_ref, bt_ref,
             o_ref, sums_ref, cnt_ref):
        i = pl.program_id(0)
        dinv = _dinv_from(dp_ref[...])
        a = a_ref[...]
        t = (a[0] + a[1] + y_ref[...]) * dinv[:, None] + b_ref[...]
        hh = _leaky(t)
        bt = bt_ref[0, 0, :]
        oh = (bt[:, None] == lax.broadcasted_iota(jnp.int32, (bn, _G), 1)
              ).astype(jnp.float32)

        @pl.when(i == 0)
        def _():
            sums_ref[...] = jnp.zeros_like(sums_ref)
            cnt_ref[...] = jnp.zeros_like(cnt_ref)

        dn = (((0,), (0,)), ((), ()))
        sums_ref[...] += lax.dot_general(oh, hh, dn,
                                         preferred_element_type=jnp.float32)
        cnt_ref[...] += lax.dot_general(oh, jnp.ones((bn, h), jnp.float32), dn,
                                        preferred_element_type=jnp.float32)

        @pl.when(i == grid - 1)
        def _():
            pooled = sums_ref[...] / jnp.maximum(cnt_ref[...], 1.0)
            o_ref[...] = jnp.dot(pooled, w_ref[...],
                                 preferred_element_type=jnp.float32) + bias_ref[0, 0]

    return pl.pallas_call(
        body,
        grid=(grid,),
        in_specs=[
            pl.BlockSpec((bn, h), lambda i: (i, 0)),
            pl.BlockSpec((2, bn, h), lambda i: (0, i, 0)),
            pl.BlockSpec((bn, 32), lambda i: (i, 0)),
            pl.BlockSpec((1, h), lambda i: (0, 0)),
            pl.BlockSpec((h, 1), lambda i: (0, 0)),
            pl.BlockSpec((1, 1), lambda i: (0, 0)),
            pl.BlockSpec((1, 1, bn), lambda i: (i, 0, 0)),
        ],
        out_specs=pl.BlockSpec((_G, 1), lambda i: (0, 0)),
        out_shape=jax.ShapeDtypeStruct((_G, 1), jnp.float32),
        scratch_shapes=[
            pltpu.VMEM((_G, h), jnp.float32),
            pltpu.VMEM((_G, h), jnp.float32),
        ],
    )(y2, agg2, degp, b2, wfc, bfc, batch3)


def kernel(x, edge_index, batch, W1, b1, W2, b2, Wfc, bfc):
    n, d = x.shape
    h = W1.shape[1]
    e = edge_index.shape[1]
    bn = 1000
    nw, k = 32, 128
    gsz = 32
    slow_core = 0
    cn, cs = 3 * gsz, 2 * gsz

    nfast = 16 * cn * k
    nslow = 16 * cs * k
    ef = min(e, nfast)
    src = edge_index[0]
    dst = edge_index[1]

    def asym(flat, filler):
        a_fast = jnp.concatenate(
            [flat[:ef], jnp.full((nfast - ef,), filler, jnp.int32)]
        ).reshape(16, cn, k)
        a_slow = jnp.concatenate(
            [flat[ef:], jnp.full((nfast + nslow - e - (nfast - ef),), filler,
                                 jnp.int32)]
        ).reshape(16, cs, k)
        a_slow = jnp.concatenate(
            [a_slow, jnp.full((16, cn - cs, k), filler, jnp.int32)], axis=1)
        pair = (a_slow, a_fast) if slow_core == 0 else (a_fast, a_slow)
        return jnp.stack(pair, axis=1).reshape(nw, cn, k)

    srcp = asym(src, 0)
    dstp = asym(dst, n)
    dst_sym = jnp.concatenate(
        [dst, jnp.full((nfast + nslow - e,), n, jnp.int32)]).reshape(nw, -1)

    npad = -(-n // 128) * 128
    zeros_nd = jnp.zeros((npad, d), jnp.float32)

    degp = _sc_degree(dst_sym, npad).T
    y1 = _tc_first(x, W1, degp, bn)
    agg1 = _sc_edge_agg(y1, srcp, dstp, zeros_nd, slow_core, cs // gsz)
    y2 = _tc_mid(y1, agg1, degp, b1.reshape(1, h), W2, bn)
    agg2 = _sc_edge_agg(y2, srcp, dstp, zeros_nd, slow_core, cs // gsz)
    out = _tc_last(y2, agg2, degp, b2.reshape(1, h), Wfc, bfc.reshape(1, 1),
                   batch.reshape(n // bn, 1, bn), bn)
    return out.reshape(-1)


# speedup vs baseline: 1.1308x; 1.1308x over previous
import functools

import jax
import jax.numpy as jnp
from jax import lax
from jax.experimental import pallas as pl
from jax.experimental.pallas import tpu as pltpu
from jax.experimental.pallas import tpu_sc as plsc

_NEG = 0.2
_G = 64


def _leaky(t):
    return jnp.where(t >= 0, t, _NEG * t)


def _sc_degree(dst2, npad):
    nw, ept = dst2.shape
    mesh = plsc.VectorSubcoreMesh(core_axis_name="c", subcore_axis_name="s")

    @functools.partial(
        pl.kernel,
        mesh=mesh,
        compiler_params=pltpu.CompilerParams(needs_layout_passes=False),
        out_type=jax.ShapeDtypeStruct((nw, npad), jnp.float32),
        scratch_types=[
            pltpu.VMEM((ept,), jnp.int32),
            pltpu.VMEM((npad,), jnp.float32),
        ],
    )
    def deg_kernel(dst_hbm, out_hbm, dst_v, deg_v):
        c = lax.axis_index("c")
        s = lax.axis_index("s")
        wid = s * 2 + c
        pltpu.sync_copy(dst_hbm.at[wid], dst_v)

        zero16 = jnp.zeros((16,), jnp.float32)

        def zbody(ri, carry):
            deg_v[pl.ds(ri * 16, 16)] = zero16
            return carry

        lax.fori_loop(0, npad // 16, zbody, 0)

        one16 = jnp.ones((16,), jnp.float32)

        def body(gi, carry):
            idx = dst_v[pl.ds(gi * 16, 16)]
            plsc.addupdate_scatter(deg_v, [idx], one16)
            return carry

        lax.fori_loop(0, ept // 16, body, 0)
        pltpu.sync_copy(deg_v, out_hbm.at[wid])

    return deg_kernel(dst2)


def _sc_edge_agg(y, src3, dst3, zeros_nd, slow_core, ngroups_slow):
    n, d = y.shape
    nw, c_chunks, k = src3.shape
    npad = zeros_nd.shape[0]
    rpt = npad // 16
    gsz = 40
    npairs = gsz // 2
    ngroups_fast = c_chunks // gsz
    mesh = plsc.VectorSubcoreMesh(core_axis_name="c", subcore_axis_name="s")

    @functools.partial(
        pl.kernel,
        mesh=mesh,
        out_type=jax.ShapeDtypeStruct((2 * npad, d), jnp.float32),
        scratch_types=[
            pltpu.VMEM((gsz, k), jnp.int32),
            pltpu.VMEM((gsz, k), jnp.int32),
            pltpu.VMEM((2, k, d), jnp.float32),
            pltpu.VMEM_SHARED((npad, d), jnp.float32),
            pltpu.SemaphoreType.DMA,
            pltpu.SemaphoreType.DMA,
            pltpu.SemaphoreType.DMA,
            pltpu.SemaphoreType.DMA,
        ],
    )
    def agg_kernel(y_hbm, src_hbm, dst_hbm, z_hbm, out_hbm,
                   src_v, dst_v, rows_v, agg_sh, g0, g1, s0, s1):
        c = lax.axis_index("c")
        s = lax.axis_index("s")
        wid = s * 2 + c
        ngroups_me = lax.select(c == slow_core,
                                jnp.int32(ngroups_slow),
                                jnp.int32(ngroups_fast))
        pltpu.sync_copy(z_hbm.at[pl.ds(s * rpt, rpt)],
                        agg_sh.at[pl.ds(s * rpt, rpt)])
        plsc.subcore_barrier()

        def gather(ci, buf, sem):
            pltpu.async_copy(y_hbm.at[src_v.at[ci]], rows_v.at[buf], sem)

        def wait_gather(buf, sem):
            pltpu.make_async_copy(y_hbm.at[pl.ds(0, k)], rows_v.at[buf],
                                  sem).wait()

        def scatter(ci, buf, sem):
            pltpu.async_copy(rows_v.at[buf], agg_sh.at[dst_v.at[ci]], sem,
                             add=True)

        def wait_scatter(buf, sem):
            pltpu.make_async_copy(rows_v.at[buf], agg_sh.at[dst_v.at[0]],
                                  sem).wait()

        def group(g, carry):
            @pl.when(g > 0)
            def _():
                wait_scatter(0, s0)
                wait_scatter(1, s1)

            pltpu.sync_copy(src_hbm.at[wid, pl.ds(g * gsz, gsz)], src_v)
            pltpu.sync_copy(dst_hbm.at[wid, pl.ds(g * gsz, gsz)], dst_v)
            gather(0, 0, g0)
            gather(1, 1, g1)

            def body(p, carry2):
                c0 = 2 * p
                wait_gather(0, g0)
                scatter(c0, 0, s0)
                wait_gather(1, g1)
                scatter(c0 + 1, 1, s1)

                @pl.when(p < npairs - 1)
                def _():
                    wait_scatter(0, s0)
                    gather(c0 + 2, 0, g0)
                    wait_scatter(1, s1)
                    gather(c0 + 3, 1, g1)

                return carry2

            lax.fori_loop(0, npairs, body, 0)
            return carry

        lax.fori_loop(0, ngroups_me, group, 0)
        wait_scatter(0, s0)
        wait_scatter(1, s1)
        plsc.subcore_barrier()
        pltpu.sync_copy(agg_sh.at[pl.ds(s * rpt, rpt)],
                        out_hbm.at[pl.ds(c * npad + s * rpt, rpt)])

    return agg_kernel(y, src3, dst3, zeros_nd).reshape(2, npad, d)


def _dinv_from(dp):
    return lax.rsqrt(1.0 + jnp.sum(dp, axis=1))


def _tc_first(x, w1, degp, bn):
    n, d = x.shape
    h = w1.shape[1]

    def body(x_ref, w_ref, dp_ref, o_ref):
        dinv = _dinv_from(dp_ref[...])
        xw = jnp.dot(x_ref[...], w_ref[...], preferred_element_type=jnp.float32)
        o_ref[...] = xw * dinv[:, None]

    return pl.pallas_call(
        body,
        grid=(n // bn,),
        in_specs=[
            pl.BlockSpec((bn, d), lambda i: (i, 0)),
            pl.BlockSpec((d, h), lambda i: (0, 0)),
            pl.BlockSpec((bn, 32), lambda i: (i, 0)),
        ],
        out_specs=pl.BlockSpec((bn, h), lambda i: (i, 0)),
        out_shape=jax.ShapeDtypeStruct((n, h), jnp.float32),
    )(x, w1, degp)


def _tc_mid(y1, agg1, degp, b1, w2, bn):
    n, h = y1.shape

    def body(y_ref, a_ref, dp_ref, b_ref, w_ref, o_ref):
        dinv = _dinv_from(dp_ref[...])
        a = a_ref[...]
        t = (a[0] + a[1] + y_ref[...]) * dinv[:, None] + b_ref[...]
        hh = _leaky(t)
        o_ref[...] = jnp.dot(hh, w_ref[...],
                             preferred_element_type=jnp.float32) * dinv[:, None]

    return pl.pallas_call(
        body,
        grid=(n // bn,),
        in_specs=[
            pl.BlockSpec((bn, h), lambda i: (i, 0)),
            pl.BlockSpec((2, bn, h), lambda i: (0, i, 0)),
            pl.BlockSpec((bn, 32), lambda i: (i, 0)),
            pl.BlockSpec((1, h), lambda i: (0, 0)),
            pl.BlockSpec((h, h), lambda i: (0, 0)),
        ],
        out_specs=pl.BlockSpec((bn, h), lambda i: (i, 0)),
        out_shape=jax.ShapeDtypeStruct((n, h), jnp.float32),
    )(y1, agg1, degp, b1, w2)


def _tc_last(y2, agg2, degp, b2, wfc, bfc, batch3, bn):
    n, h = y2.shape
    grid = n // bn

    def body(y_ref, a_ref, dp_ref, b_ref, w_ref, bias_ref, bt_ref,
             o_ref, sums_ref, cnt_ref):
        i = pl.program_id(0)
        dinv = _dinv_from(dp_ref[...])
        a = a_ref[...]
        t = (a[0] + a[1] + y_ref[...]) * dinv[:, None] + b_ref[...]
        hh = _leaky(t)
        bt = bt_ref[0, 0, :]
        oh = (bt[:, None] == lax.broadcasted_iota(jnp.int32, (bn, _G), 1)
              ).astype(jnp.float32)

        @pl.when(i == 0)
        def _():
            sums_ref[...] = jnp.zeros_like(sums_ref)
            cnt_ref[...] = jnp.zeros_like(cnt_ref)

        dn = (((0,), (0,)), ((), ()))
        sums_ref[...] += lax.dot_general(oh, hh, dn,
                                         preferred_element_type=jnp.float32)
        cnt_ref[...] += lax.dot_general(oh, jnp.ones((bn, h), jnp.float32), dn,
                                        preferred_element_type=jnp.float32)

        @pl.when(i == grid - 1)
        def _():
            pooled = sums_ref[...] / jnp.maximum(cnt_ref[...], 1.0)
            o_ref[...] = jnp.dot(pooled, w_ref[...],
                                 preferred_element_type=jnp.float32) + bias_ref[0, 0]

    return pl.pallas_call(
        body,
        grid=(grid,),
        in_specs=[
            pl.BlockSpec((bn, h), lambda i: (i, 0)),
            pl.BlockSpec((2, bn, h), lambda i: (0, i, 0)),
            pl.BlockSpec((bn, 32), lambda i: (i, 0)),
            pl.BlockSpec((1, h), lambda i: (0, 0)),
            pl.BlockSpec((h, 1), lambda i: (0, 0)),
            pl.BlockSpec((1, 1), lambda i: (0, 0)),
            pl.BlockSpec((1, 1, bn), lambda i: (i, 0, 0)),
        ],
        out_specs=pl.BlockSpec((_G, 1), lambda i: (0, 0)),
        out_shape=jax.ShapeDtypeStruct((_G, 1), jnp.float32),
        scratch_shapes=[
            pltpu.VMEM((_G, h), jnp.float32),
            pltpu.VMEM((_G, h), jnp.float32),
        ],
    )(y2, agg2, degp, b2, wfc, bfc, batch3)


def kernel(x, edge_index, batch, W1, b1, W2, b2, Wfc, bfc):
    n, d = x.shape
    h = W1.shape[1]
    e = edge_index.shape[1]
    bn = 1000
    nw, k = 32, 128
    gsz = 40
    slow_core = 0
    cn, cs = 3 * gsz, 1 * gsz

    nfast = 16 * cn * k
    nslow = 16 * cs * k
    ef = min(e, nfast)
    src = edge_index[0]
    dst = edge_index[1]

    def asym(flat, filler):
        a_fast = jnp.concatenate(
            [flat[:ef], jnp.full((nfast - ef,), filler, jnp.int32)]
        ).reshape(16, cn, k)
        a_slow = jnp.concatenate(
            [flat[ef:], jnp.full((nfast + nslow - e - (nfast - ef),), filler,
                                 jnp.int32)]
        ).reshape(16, cs, k)
        a_slow = jnp.concatenate(
            [a_slow, jnp.full((16, cn - cs, k), filler, jnp.int32)], axis=1)
        pair = (a_slow, a_fast) if slow_core == 0 else (a_fast, a_slow)
        return jnp.stack(pair, axis=1).reshape(nw, cn, k)

    srcp = asym(src, 0)
    dstp = asym(dst, n)
    dst_sym = jnp.concatenate(
        [dst, jnp.full((nfast + nslow - e,), n, jnp.int32)]).reshape(nw, -1)

    npad = -(-n // 128) * 128
    zeros_nd = jnp.zeros((npad, d), jnp.float32)

    degp = _sc_degree(dst_sym, npad).T
    y1 = _tc_first(x, W1, degp, bn)
    agg1 = _sc_edge_agg(y1, srcp, dstp, zeros_nd, slow_core, cs // gsz)
    y2 = _tc_mid(y1, agg1, degp, b1.reshape(1, h), W2, bn)
    agg2 = _sc_edge_agg(y2, srcp, dstp, zeros_nd, slow_core, cs // gsz)
    out = _tc_last(y2, agg2, degp, b2.reshape(1, h), Wfc, bfc.reshape(1, 1),
                   batch.reshape(n // bn, 1, bn), bn)
    return out.reshape(-1)
